# baseline (speedup 1.0000x reference)
import functools

import jax
import jax.numpy as jnp
from jax.experimental import pallas as pl
from jax.experimental.pallas import tpu as pltpu

B, T, D_IN = 32, 128, 1024
D_H1 = 512
H = 64
K = 1024
B_BLK = 8
N_BLK = B // B_BLK

_PREC = jax.lax.Precision.DEFAULT


def _dot(a, b, precision=_PREC):
    return jax.lax.dot_general(a, b, (((1,), (0,)), ((), ())),
                               precision=precision,
                               preferred_element_type=jnp.float32)


def _fused_fwd_kernel(x_ref, w1t_ref, b1_ref, w2t_ref, b2_ref,
                      wc0t_ref, wc1t_ref, wc2t_ref, bc_ref,
                      cbt_ref, cb_ref,
                      tok_ref, q_ref, feat_ref, commit_ref):
    i = pl.program_id(0)
    rows = B_BLK * T

    xb = x_ref[...].reshape(rows, D_IN)
    h1 = jnp.maximum(_dot(xb, w1t_ref[...]) + b1_ref[...], 0.0)
    f = jnp.maximum(_dot(h1, w2t_ref[...]) + b2_ref[...], 0.0)

    f3 = f.reshape(B_BLK, T, H)
    zpad = jnp.zeros((B_BLK, 1, H), dtype=jnp.float32)
    fprev = jnp.concatenate([zpad, f3[:, :-1, :]], axis=1).reshape(rows, H)
    fnext = jnp.concatenate([f3[:, 1:, :], zpad], axis=1).reshape(rows, H)
    fcat = jnp.concatenate([fprev, f, fnext], axis=1)
    wcat = jnp.concatenate(
        [wc0t_ref[...], wc1t_ref[...], wc2t_ref[...]], axis=0)
    feats = _dot(fcat, wcat) + bc_ref[...]

    cbt = cbt_ref[...]
    a2 = jnp.sum(feats * feats, axis=1, keepdims=True)
    m = _dot(feats, cbt)
    cb2 = jnp.sum(cbt * cbt, axis=0, keepdims=True)
    d2 = (a2 - 2.0 * m) + cb2
    dist = jnp.sqrt(jnp.maximum(d2, 0.0))
    dmin = jnp.min(dist, axis=1, keepdims=True)
    lane = jax.lax.broadcasted_iota(jnp.int32, (rows, K), 1)
    idx = jnp.min(jnp.where(dist <= dmin, lane, K), axis=1)

    tok_ref[...] = idx.reshape(B_BLK, T)

    onehot = (jax.lax.broadcasted_iota(jnp.int32, (rows, K), 1)
              == idx[:, None]).astype(jnp.float32)
    q = _dot(onehot, cb_ref[...])
    q_ref[...] = q.reshape(B_BLK, T, H)
    feat_ref[...] = feats.reshape(B_BLK, T, H)

    dq = feats - q
    part = jnp.sum(dq * dq, axis=(0, 1), keepdims=True)

    @pl.when(i == 0)
    def _():
        commit_ref[...] = jnp.zeros((1, 1), jnp.float32)

    commit_ref[...] += part

    @pl.when(i == N_BLK - 1)
    def _():
        commit_ref[...] = commit_ref[...] / (B * T * H)


def _gru_kernel(q_ref, f_ref, wicat_ref, bicat_ref, whcat_ref, bhcat_ref,
                ctx_ref, gi_ref):
    Tm = T - 1
    rows = Tm * B
    qflat = q_ref[...].reshape(rows, H)
    gi_ref[...] = (_dot(qflat, wicat_ref[...])
                   + bicat_ref[...]).reshape(Tm, B, 384)

    whcat = whcat_ref[...]
    bhcat = bhcat_ref[...]

    def step(t, carry):
        h, acc = carry
        gh = _dot(h, whcat) + bhcat
        gi = gi_ref[pl.ds(t, 1)].reshape(B, 384)
        r = jax.nn.sigmoid(gi[:, 0:H] + gh[:, 0:H])
        z = jax.nn.sigmoid(gi[:, 128:128 + H] + gh[:, 128:128 + H])
        n = jnp.tanh(gi[:, 256:256 + H] + r * gh[:, 256:256 + H])
        h_new = (1.0 - z) * n + z * h
        ft = f_ref[pl.ds(t, 1)].reshape(B, H)
        d = h_new - ft
        return h_new, acc + d * d

    h0 = jnp.zeros((B, H), dtype=jnp.float32)
    acc0 = jnp.zeros((B, H), dtype=jnp.float32)
    _, acc = jax.lax.fori_loop(0, Tm, step, (h0, acc0), unroll=4)
    ctx_ref[...] = jnp.sum(acc, axis=(0, 1), keepdims=True) / (B * Tm * H)


@functools.partial(jax.jit, static_argnames=())
def kernel(x, W1, b1, W2, b2, Wc, bc, codebook, W_ih, W_hh, b_ih, b_hh):
    w1t = W1.T
    b1r = b1.reshape(1, D_H1)
    w2t = W2.T
    b2r = b2.reshape(1, H)
    wc0t = Wc[:, :, 0].T
    wc1t = Wc[:, :, 1].T
    wc2t = Wc[:, :, 2].T
    bcr = bc.reshape(1, H)
    cbt = codebook.T

    full = lambda shp: pl.BlockSpec(shp, lambda i: (0,) * len(shp))
    tok, quantized, features, commit = pl.pallas_call(
        _fused_fwd_kernel,
        grid=(N_BLK,),
        in_specs=[
            pl.BlockSpec((B_BLK, T, D_IN), lambda i: (i, 0, 0)),
            full((D_IN, D_H1)), full((1, D_H1)),
            full((D_H1, H)), full((1, H)),
            full((H, H)), full((H, H)), full((H, H)), full((1, H)),
            full((H, K)), full((K, H)),
        ],
        out_specs=[
            pl.BlockSpec((B_BLK, T), lambda i: (i, 0)),
            pl.BlockSpec((B_BLK, T, H), lambda i: (i, 0, 0)),
            pl.BlockSpec((B_BLK, T, H), lambda i: (i, 0, 0)),
            pl.BlockSpec((1, 1), lambda i: (0, 0)),
        ],
        out_shape=[
            jax.ShapeDtypeStruct((B, T), jnp.int32),
            jax.ShapeDtypeStruct((B, T, H), jnp.float32),
            jax.ShapeDtypeStruct((B, T, H), jnp.float32),
            jax.ShapeDtypeStruct((1, 1), jnp.float32),
        ],
    )(x, w1t, b1r, w2t, b2r, wc0t, wc1t, wc2t, bcr, cbt, codebook)

    qT = jnp.swapaxes(quantized, 0, 1)[: T - 1]
    fT = jnp.swapaxes(features, 0, 1)[1:]

    zpadw = jnp.zeros((H, 128 - H), jnp.float32)
    wicat = jnp.concatenate(
        [W_ih[0:H].T, zpadw, W_ih[H:2 * H].T, zpadw, W_ih[2 * H:3 * H].T,
         zpadw], axis=1)
    whcat = jnp.concatenate(
        [W_hh[0:H].T, zpadw, W_hh[H:2 * H].T, zpadw, W_hh[2 * H:3 * H].T,
         zpadw], axis=1)
    zpadb = jnp.zeros((1, 128 - H), jnp.float32)
    bicat = jnp.concatenate(
        [b_ih[0:H].reshape(1, H), zpadb, b_ih[H:2 * H].reshape(1, H), zpadb,
         b_ih[2 * H:3 * H].reshape(1, H), zpadb], axis=1)
    bhcat = jnp.concatenate(
        [b_hh[0:H].reshape(1, H), zpadb, b_hh[H:2 * H].reshape(1, H), zpadb,
         b_hh[2 * H:3 * H].reshape(1, H), zpadb], axis=1)

    ctx = pl.pallas_call(
        _gru_kernel,
        scratch_shapes=[
            pltpu.VMEM((T - 1, B, 384), jnp.float32),
        ],
        out_shape=jax.ShapeDtypeStruct((1, 1), jnp.float32),
    )(qT, fT, wicat, bicat, whcat, bhcat)

    commitment_loss = commit[0, 0]
    codebook_loss = commitment_loss
    context_loss = ctx[0, 0]
    vq_loss = commitment_loss + 0.25 * codebook_loss + 0.1 * context_loss
    return (tok, quantized, commitment_loss, codebook_loss,
            context_loss, vq_loss)


# speedup vs baseline: 1.0043x; 1.0043x over previous
import functools

import jax
import jax.numpy as jnp
from jax.experimental import pallas as pl
from jax.experimental.pallas import tpu as pltpu

B, T, D_IN = 32, 128, 1024
D_H1 = 512
H = 64
K = 1024
B_BLK = 8
N_BLK = B // B_BLK

_PREC = jax.lax.Precision.DEFAULT


def _dot(a, b, precision=_PREC):
    return jax.lax.dot_general(a, b, (((1,), (0,)), ((), ())),
                               precision=precision,
                               preferred_element_type=jnp.float32)


def _dott(a, b, precision=_PREC):
    return jax.lax.dot_general(a, b, (((1,), (1,)), ((), ())),
                               precision=precision,
                               preferred_element_type=jnp.float32)


def _fused_fwd_kernel(x_ref, w1_ref, b1_ref, w2_ref, b2_ref,
                      wck_ref, bc_ref, cb_ref,
                      tok_ref, q_ref, feat_ref, commit_ref):
    i = pl.program_id(0)
    rows = B_BLK * T

    xb = x_ref[...].reshape(rows, D_IN)
    h1 = jnp.maximum(_dott(xb, w1_ref[...]) + b1_ref[...], 0.0)
    f = jnp.maximum(_dott(h1, w2_ref[...]) + b2_ref[...], 0.0)

    f3 = f.reshape(B_BLK, T, H)
    zpad = jnp.zeros((B_BLK, 1, H), dtype=jnp.float32)
    fprev = jnp.concatenate([zpad, f3[:, :-1, :]], axis=1).reshape(rows, H)
    fnext = jnp.concatenate([f3[:, 1:, :], zpad], axis=1).reshape(rows, H)
    fcat = jnp.concatenate([fprev, f, fnext], axis=1)
    feats = _dott(fcat, wck_ref[...]) + bc_ref[...]

    cb = cb_ref[...]
    a2 = jnp.sum(feats * feats, axis=1, keepdims=True)
    m = _dott(feats, cb)
    cb2 = _dott(jnp.ones((1, H), jnp.float32), cb * cb)
    d2 = (a2 - 2.0 * m) + cb2
    dist = jnp.sqrt(jnp.maximum(d2, 0.0))
    dmin = jnp.min(dist, axis=1, keepdims=True)
    lane = jax.lax.broadcasted_iota(jnp.int32, (rows, K), 1)
    idx = jnp.min(jnp.where(dist <= dmin, lane, K), axis=1)

    tok_ref[...] = idx.reshape(B_BLK, T)

    onehot = (jax.lax.broadcasted_iota(jnp.int32, (rows, K), 1)
              == idx[:, None]).astype(jnp.float32)
    q = _dot(onehot, cb_ref[...])
    q_ref[...] = q.reshape(B_BLK, T, H)
    feat_ref[...] = feats.reshape(B_BLK, T, H)

    dq = feats - q
    part = jnp.sum(dq * dq, axis=(0, 1), keepdims=True)

    @pl.when(i == 0)
    def _():
        commit_ref[...] = jnp.zeros((1, 1), jnp.float32)

    commit_ref[...] += part

    @pl.when(i == N_BLK - 1)
    def _():
        commit_ref[...] = commit_ref[...] / (B * T * H)


def _gru_kernel(q_ref, f_ref, wicat_ref, bicat_ref, whcat_ref, bhcat_ref,
                ctx_ref, gi_ref):
    Tm = T - 1
    rows = Tm * B
    qflat = q_ref[...].reshape(rows, H)
    gi_ref[...] = (_dot(qflat, wicat_ref[...])
                   + bicat_ref[...]).reshape(Tm, B, 384)

    whcat = whcat_ref[...]
    bhcat = bhcat_ref[...]

    def step(t, carry):
        h, acc = carry
        gh = _dot(h, whcat) + bhcat
        gi = gi_ref[pl.ds(t, 1)].reshape(B, 384)
        r = jax.nn.sigmoid(gi[:, 0:H] + gh[:, 0:H])
        z = jax.nn.sigmoid(gi[:, 128:128 + H] + gh[:, 128:128 + H])
        n = jnp.tanh(gi[:, 256:256 + H] + r * gh[:, 256:256 + H])
        h_new = (1.0 - z) * n + z * h
        ft = f_ref[pl.ds(t, 1)].reshape(B, H)
        d = h_new - ft
        return h_new, acc + d * d

    h0 = jnp.zeros((B, H), dtype=jnp.float32)
    acc0 = jnp.zeros((B, H), dtype=jnp.float32)
    _, acc = jax.lax.fori_loop(0, Tm, step, (h0, acc0), unroll=4)
    ctx_ref[...] = jnp.sum(acc, axis=(0, 1), keepdims=True) / (B * Tm * H)


@functools.partial(jax.jit, static_argnames=())
def kernel(x, W1, b1, W2, b2, Wc, bc, codebook, W_ih, W_hh, b_ih, b_hh):
    wck = Wc.transpose(0, 2, 1).reshape(H, 3 * H)
    bcr = bc.reshape(1, H)

    full = lambda shp: pl.BlockSpec(shp, lambda i: (0,) * len(shp))
    tok, quantized, features, commit = pl.pallas_call(
        _fused_fwd_kernel,
        grid=(N_BLK,),
        in_specs=[
            pl.BlockSpec((B_BLK, T, D_IN), lambda i: (i, 0, 0)),
            full((D_H1, D_IN)), full((1, D_H1)),
            full((H, D_H1)), full((1, H)),
            full((H, 3 * H)), full((1, H)),
            full((K, H)),
        ],
        out_specs=[
            pl.BlockSpec((B_BLK, T), lambda i: (i, 0)),
            pl.BlockSpec((B_BLK, T, H), lambda i: (i, 0, 0)),
            pl.BlockSpec((B_BLK, T, H), lambda i: (i, 0, 0)),
            pl.BlockSpec((1, 1), lambda i: (0, 0)),
        ],
        out_shape=[
            jax.ShapeDtypeStruct((B, T), jnp.int32),
            jax.ShapeDtypeStruct((B, T, H), jnp.float32),
            jax.ShapeDtypeStruct((B, T, H), jnp.float32),
            jax.ShapeDtypeStruct((1, 1), jnp.float32),
        ],
    )(x, W1, b1.reshape(1, D_H1), W2, b2.reshape(1, H), wck, bcr, codebook)

    qT = jnp.swapaxes(quantized, 0, 1)[: T - 1]
    fT = jnp.swapaxes(features, 0, 1)[1:]

    zpadw = jnp.zeros((H, 128 - H), jnp.float32)
    wicat = jnp.concatenate(
        [W_ih[0:H].T, zpadw, W_ih[H:2 * H].T, zpadw, W_ih[2 * H:3 * H].T,
         zpadw], axis=1)
    whcat = jnp.concatenate(
        [W_hh[0:H].T, zpadw, W_hh[H:2 * H].T, zpadw, W_hh[2 * H:3 * H].T,
         zpadw], axis=1)
    zpadb = jnp.zeros((1, 128 - H), jnp.float32)
    bicat = jnp.concatenate(
        [b_ih[0:H].reshape(1, H), zpadb, b_ih[H:2 * H].reshape(1, H), zpadb,
         b_ih[2 * H:3 * H].reshape(1, H), zpadb], axis=1)
    bhcat = jnp.concatenate(
        [b_hh[0:H].reshape(1, H), zpadb, b_hh[H:2 * H].reshape(1, H), zpadb,
         b_hh[2 * H:3 * H].reshape(1, H), zpadb], axis=1)

    ctx = pl.pallas_call(
        _gru_kernel,
        scratch_shapes=[
            pltpu.VMEM((T - 1, B, 384), jnp.float32),
        ],
        out_shape=jax.ShapeDtypeStruct((1, 1), jnp.float32),
    )(qT, fT, wicat, bicat, whcat, bhcat)

    commitment_loss = commit[0, 0]
    codebook_loss = commitment_loss
    context_loss = ctx[0, 0]
    vq_loss = commitment_loss + 0.25 * codebook_loss + 0.1 * context_loss
    return (tok, quantized, commitment_loss, codebook_loss,
            context_loss, vq_loss)
